# baseline (speedup 1.0000x reference)
import functools

import jax
import jax.numpy as jnp
from jax import lax
from jax.experimental import pallas as pl
from jax.experimental.pallas import tpu as pltpu
from jax.experimental.pallas import tpu_sc as plsc

B, C, N, K = 4, 64, 2048, 16
CIN = 2 * C
CEXP = 2 * CIN
NB = 256
NB2 = 256
BF = jnp.bfloat16
_TW = 128

def _knn_body(x_ref, xt_ref, idx_ref, tab_ref):
    R = xt_ref[0]
    X = x_ref[0]
    inner = lax.dot_general(R, X, (((1,), (0,)), ((), ())),
                            preferred_element_type=jnp.float32)
    sq_all = jnp.sum(X * X, axis=0, keepdims=True)
    sq_r = jnp.sum(R * R, axis=1, keepdims=True)
    d = jnp.maximum(sq_r + sq_all - 2.0 * inner, 0.0)
    col = lax.broadcasted_iota(jnp.int32, d.shape, 1)
    pi = (lax.bitcast_convert_type(d, jnp.int32) & ~jnp.int32(N - 1)) | col
    p = lax.bitcast_convert_type(pi, jnp.float32)
    sentinel = jnp.float32(jnp.inf)
    m = jnp.min(p, axis=1, keepdims=True)
    for t in range(1, K + 1):
        m = jnp.min(jnp.where(p > m, p, sentinel), axis=1, keepdims=True)
        mi = lax.bitcast_convert_type(m[:, 0], jnp.int32)
        idx_ref[:, t - 1] = mi & jnp.int32(N - 1)
    tab_ref[...] = R


@functools.lru_cache(maxsize=None)
def _knn_call(b):
    return pl.pallas_call(
        _knn_body,
        grid=(N // NB,),
        in_specs=[
            pl.BlockSpec((1, C, N), lambda i: (b, 0, 0)),
            pl.BlockSpec((1, NB, C), lambda i: (b, i, 0)),
        ],
        out_specs=[
            pl.BlockSpec((NB, K), lambda i: (i, 0)),
            pl.BlockSpec((NB, C), lambda i: (i, 0)),
        ],
        out_shape=[
            jax.ShapeDtypeStruct((N, K), jnp.int32),
            jax.ShapeDtypeStruct((N, C), jnp.float32),
        ],
    )


_NW = 32
_NC = 2
_CHUNK = 128
_ROWS = N * K
_PER_W = _ROWS // _NW
_N_CHUNKS = _PER_W // _CHUNK


def _gather_body(table_hbm, idx_hbm, out_hbm, idx_v, rows0, rows1, sg, sw):
    wid = lax.axis_index("s") * _NC + lax.axis_index("c")
    base = pl.multiple_of(wid * _PER_W, _CHUNK)
    pltpu.sync_copy(idx_hbm.at[pl.ds(base, _PER_W)], idx_v)

    def body(i, carry):
        off0 = i * 2 * _CHUNK
        off1 = off0 + _CHUNK
        g0 = pltpu.async_copy(
            table_hbm.at[idx_v.at[pl.ds(off0, _CHUNK)]], rows0, sg)
        g1 = pltpu.async_copy(
            table_hbm.at[idx_v.at[pl.ds(off1, _CHUNK)]], rows1, sg)
        g0.wait()
        w0 = pltpu.async_copy(rows0, out_hbm.at[pl.ds(base + off0, _CHUNK)], sw)
        g1.wait()
        w1 = pltpu.async_copy(rows1, out_hbm.at[pl.ds(base + off1, _CHUNK)], sw)
        w0.wait()
        w1.wait()
        return carry

    lax.fori_loop(0, _N_CHUNKS // 2, body, 0)


@functools.lru_cache(maxsize=1)
def _gather_call():
    return pl.kernel(
        _gather_body,
        out_type=jax.ShapeDtypeStruct((_ROWS, C), jnp.float32),
        compiler_params=pltpu.CompilerParams(use_tc_tiling_on_sc=False),
        mesh=plsc.VectorSubcoreMesh(core_axis_name="c", subcore_axis_name="s"),
        scratch_types=[
            pltpu.VMEM((_PER_W,), jnp.int32),
            pltpu.VMEM((_CHUNK, C), jnp.float32),
            pltpu.VMEM((_CHUNK, C), jnp.float32),
            pltpu.SemaphoreType.DMA,
            pltpu.SemaphoreType.DMA,
        ],
    )


def _mlp_body(g_ref, xt_ref, x_ref, w1d_ref, b1_ref, w1b_ref, w2_ref, b2_ref,
              w3_ref, b3_ref, o_ref):
    R = xt_ref[0]
    V = lax.dot_general(R, w1d_ref[...], (((1,), (1,)), ((), ())),
                        preferred_element_type=jnp.float32) + b1_ref[...]
    G = g_ref[...].reshape(NB2 * K, C).astype(BF)
    A1 = lax.dot_general(G, w1b_ref[...], (((1,), (1,)), ((), ())),
                         preferred_element_type=jnp.float32)
    Z1 = jnp.maximum(A1.reshape(NB2, K, CEXP) + V[:, None, :], 0.0)
    Z1 = Z1.reshape(NB2 * K, CEXP).astype(BF)
    Z2 = jnp.maximum(
        lax.dot_general(Z1, w2_ref[...], (((1,), (1,)), ((), ())),
                        preferred_element_type=jnp.float32) + b2_ref[...], 0.0)
    Z3 = lax.dot_general(Z2.astype(BF), w3_ref[...], (((1,), (1,)), ((), ())),
                         preferred_element_type=jnp.float32) + b3_ref[...]
    res = jnp.max(Z3.reshape(NB2, K, C), axis=1)
    eye = (lax.broadcasted_iota(jnp.int32, (C, C), 0) ==
           lax.broadcasted_iota(jnp.int32, (C, C), 1)).astype(jnp.float32)
    resT = lax.dot_general(eye, res, (((1,), (1,)), ((), ())),
                           preferred_element_type=jnp.float32)
    o_ref[...] = resT + x_ref[0]


@functools.lru_cache(maxsize=None)
def _mlp_call(b):
    return pl.pallas_call(
        _mlp_body,
        grid=(N // NB2,),
        in_specs=[
            pl.BlockSpec((NB2, K, C), lambda i: (i, 0, 0)),
            pl.BlockSpec((1, NB2, C), lambda i: (b, i, 0)),
            pl.BlockSpec((1, C, NB2), lambda i: (b, 0, i)),
            pl.BlockSpec((CEXP, C), lambda i: (0, 0)),
            pl.BlockSpec((1, CEXP), lambda i: (0, 0)),
            pl.BlockSpec((CEXP, C), lambda i: (0, 0)),
            pl.BlockSpec((CEXP, CEXP), lambda i: (0, 0)),
            pl.BlockSpec((1, CEXP), lambda i: (0, 0)),
            pl.BlockSpec((C, CEXP), lambda i: (0, 0)),
            pl.BlockSpec((1, C), lambda i: (0, 0)),
        ],
        out_specs=pl.BlockSpec((C, NB2), lambda i: (0, i)),
        out_shape=jax.ShapeDtypeStruct((C, N), jnp.float32),
    )


def kernel(input, W1, b1, W2, b2, W3, b3):
    x = input
    xt = jnp.transpose(x, (0, 2, 1))
    w1a, w1b = W1[:, :C], W1[:, C:]
    w1d = w1a - w1b
    b1r = b1.reshape(1, CEXP)
    b2r = b2.reshape(1, CEXP)
    b3r = b3.reshape(1, C)
    w1b_bf = w1b.astype(BF)
    w2_bf = W2.astype(BF)
    w3_bf = W3.astype(BF)
    knn = [_knn_call(b)(x, xt) for b in range(B)]
    gs = [_gather_call()(tab, idx.reshape(_ROWS)) for idx, tab in knn]
    outs = [_mlp_call(b)(gs[b].reshape(N, K, C), xt, x, w1d, b1r,
                         w1b_bf, w2_bf, b2r, w3_bf, b3r) for b in range(B)]
    return jnp.stack(outs)


# speedup vs baseline: 1.1563x; 1.1563x over previous
import functools

import jax
import jax.numpy as jnp
from jax import lax
from jax.experimental import pallas as pl
from jax.experimental.pallas import tpu as pltpu
from jax.experimental.pallas import tpu_sc as plsc

B, C, N, K = 4, 64, 2048, 16
CIN = 2 * C
CEXP = 2 * CIN
NB = 256
NB2 = 256
BF = jnp.bfloat16
_TW = 128

def _knn_body(x_ref, xt_ref, idx_ref, tab_ref):
    R = xt_ref[0]
    X = x_ref[0]
    inner = lax.dot_general(R, X, (((1,), (0,)), ((), ())),
                            preferred_element_type=jnp.float32)
    sq_all = jnp.sum(X * X, axis=0, keepdims=True)
    sq_r = jnp.sum(R * R, axis=1, keepdims=True)
    d = jnp.maximum(sq_r + sq_all - 2.0 * inner, 0.0)
    col = lax.broadcasted_iota(jnp.int32, d.shape, 1)
    pi = (lax.bitcast_convert_type(d, jnp.int32) & ~jnp.int32(N - 1)) | col
    p = lax.bitcast_convert_type(pi, jnp.float32)
    sentinel = jnp.float32(jnp.inf)
    m = jnp.min(p, axis=1, keepdims=True)
    for t in range(1, K + 1):
        m = jnp.min(jnp.where(p > m, p, sentinel), axis=1, keepdims=True)
        mi = lax.bitcast_convert_type(m[:, 0], jnp.int32)
        idx_ref[:, t - 1] = mi & jnp.int32(N - 1)
    tab_ref[...] = jnp.concatenate(
        [R, jnp.zeros((NB, _TW - C), jnp.float32)], axis=1)


@functools.lru_cache(maxsize=None)
def _knn_call(b):
    return pl.pallas_call(
        _knn_body,
        grid=(N // NB,),
        in_specs=[
            pl.BlockSpec((1, C, N), lambda i: (b, 0, 0)),
            pl.BlockSpec((1, NB, C), lambda i: (b, i, 0)),
        ],
        out_specs=[
            pl.BlockSpec((NB, K), lambda i: (i, 0)),
            pl.BlockSpec((NB, _TW), lambda i: (i, 0)),
        ],
        out_shape=[
            jax.ShapeDtypeStruct((N, K), jnp.int32),
            jax.ShapeDtypeStruct((N, _TW), jnp.float32),
        ],
    )


_NW = 32
_NC = 2
_CHUNK = 128
_ROWS = N * K
_PER_W = _ROWS // _NW
_N_CHUNKS = _PER_W // _CHUNK


_NBUF = 6


def _gather_body(table_hbm, idx_hbm, out_hbm, idx_v, *rest):
    rows = rest[:_NBUF]
    sg, sw = rest[_NBUF], rest[_NBUF + 1]
    wid = lax.axis_index("s") * _NC + lax.axis_index("c")
    base = pl.multiple_of(wid * _PER_W, _CHUNK)
    pltpu.sync_copy(idx_hbm.at[pl.ds(base, _PER_W)], idx_v)

    def gath(i):
        return pltpu.async_copy(
            table_hbm.at[idx_v.at[pl.ds(i * _CHUNK, _CHUNK)]],
            rows[i % _NBUF], sg)

    def wb(i):
        return pltpu.async_copy(
            rows[i % _NBUF], out_hbm.at[pl.ds(base + i * _CHUNK, _CHUNK)], sw)

    gs = [gath(i) for i in range(_NBUF)]
    ws = []
    for i in range(_N_CHUNKS):
        gs[i].wait()
        ws.append(wb(i))
        j = i + _NBUF
        if j < _N_CHUNKS:
            ws[j - _NBUF].wait()
            gs.append(gath(j))
    for i in range(_N_CHUNKS - _NBUF, _N_CHUNKS):
        ws[i].wait()


@functools.lru_cache(maxsize=1)
def _gather_call():
    return pl.kernel(
        _gather_body,
        out_type=jax.ShapeDtypeStruct((_ROWS, _TW), jnp.float32),
        mesh=plsc.VectorSubcoreMesh(core_axis_name="c", subcore_axis_name="s"),
        scratch_types=(
            [pltpu.VMEM((_PER_W,), jnp.int32)]
            + [pltpu.VMEM((_CHUNK, _TW), jnp.float32) for _ in range(_NBUF)]
            + [pltpu.SemaphoreType.DMA, pltpu.SemaphoreType.DMA]
        ),
    )


def _mlp_body(g_ref, xt_ref, x_ref, w1d_ref, b1_ref, w1b_ref, w2_ref, b2_ref,
              w3_ref, b3_ref, o_ref):
    R = xt_ref[0]
    V = lax.dot_general(R, w1d_ref[...], (((1,), (1,)), ((), ())),
                        preferred_element_type=jnp.float32) + b1_ref[...]
    G = g_ref[:, :, :C].reshape(NB2 * K, C).astype(BF)
    A1 = lax.dot_general(G, w1b_ref[...], (((1,), (1,)), ((), ())),
                         preferred_element_type=jnp.float32)
    Z1 = jnp.maximum(A1.reshape(NB2, K, CEXP) + V[:, None, :], 0.0)
    Z1 = Z1.reshape(NB2 * K, CEXP).astype(BF)
    Z2 = jnp.maximum(
        lax.dot_general(Z1, w2_ref[...], (((1,), (1,)), ((), ())),
                        preferred_element_type=jnp.float32) + b2_ref[...], 0.0)
    Z3 = lax.dot_general(Z2.astype(BF), w3_ref[...], (((1,), (1,)), ((), ())),
                         preferred_element_type=jnp.float32) + b3_ref[...]
    res = jnp.max(Z3.reshape(NB2, K, C), axis=1)
    eye = (lax.broadcasted_iota(jnp.int32, (C, C), 0) ==
           lax.broadcasted_iota(jnp.int32, (C, C), 1)).astype(jnp.float32)
    resT = lax.dot_general(eye, res, (((1,), (1,)), ((), ())),
                           preferred_element_type=jnp.float32)
    o_ref[...] = resT + x_ref[0]


@functools.lru_cache(maxsize=None)
def _mlp_call(b):
    return pl.pallas_call(
        _mlp_body,
        grid=(N // NB2,),
        in_specs=[
            pl.BlockSpec((NB2, K, _TW), lambda i: (i, 0, 0)),
            pl.BlockSpec((1, NB2, C), lambda i: (b, i, 0)),
            pl.BlockSpec((1, C, NB2), lambda i: (b, 0, i)),
            pl.BlockSpec((CEXP, C), lambda i: (0, 0)),
            pl.BlockSpec((1, CEXP), lambda i: (0, 0)),
            pl.BlockSpec((CEXP, C), lambda i: (0, 0)),
            pl.BlockSpec((CEXP, CEXP), lambda i: (0, 0)),
            pl.BlockSpec((1, CEXP), lambda i: (0, 0)),
            pl.BlockSpec((C, CEXP), lambda i: (0, 0)),
            pl.BlockSpec((1, C), lambda i: (0, 0)),
        ],
        out_specs=pl.BlockSpec((C, NB2), lambda i: (0, i)),
        out_shape=jax.ShapeDtypeStruct((C, N), jnp.float32),
    )


def kernel(input, W1, b1, W2, b2, W3, b3):
    x = input
    xt = jnp.transpose(x, (0, 2, 1))
    w1a, w1b = W1[:, :C], W1[:, C:]
    w1d = w1a - w1b
    b1r = b1.reshape(1, CEXP)
    b2r = b2.reshape(1, CEXP)
    b3r = b3.reshape(1, C)
    w1b_bf = w1b.astype(BF)
    w2_bf = W2.astype(BF)
    w3_bf = W3.astype(BF)
    knn = [_knn_call(b)(x, xt) for b in range(B)]
    gs = [_gather_call()(tab, idx.reshape(_ROWS)) for idx, tab in knn]
    outs = [_mlp_call(b)(gs[b].reshape(N, K, _TW), xt, x, w1d, b1r,
                         w1b_bf, w2_bf, b2r, w3_bf, b3r) for b in range(B)]
    return jnp.stack(outs)


# speedup vs baseline: 1.1731x; 1.0145x over previous
import functools

import jax
import jax.numpy as jnp
from jax import lax
from jax.experimental import pallas as pl
from jax.experimental.pallas import tpu as pltpu
from jax.experimental.pallas import tpu_sc as plsc

B, C, N, K = 4, 64, 2048, 16
CIN = 2 * C
CEXP = 2 * CIN
NB = 512
NB2 = 512
BF = jnp.bfloat16
_TW = 128

def _knn_body(x_ref, xt_ref, idx_ref, tab_ref):
    R = xt_ref[0]
    X = x_ref[0]
    inner = lax.dot_general(R, X, (((1,), (0,)), ((), ())),
                            preferred_element_type=jnp.float32)
    sq_all = jnp.sum(X * X, axis=0, keepdims=True)
    sq_r = jnp.sum(R * R, axis=1, keepdims=True)
    d = jnp.maximum(sq_r + sq_all - 2.0 * inner, 0.0)
    col = lax.broadcasted_iota(jnp.int32, d.shape, 1)
    pi = (lax.bitcast_convert_type(d, jnp.int32) & ~jnp.int32(N - 1)) | col
    p = lax.bitcast_convert_type(pi, jnp.float32)
    sentinel = jnp.float32(jnp.inf)
    m = jnp.min(p, axis=1, keepdims=True)
    for t in range(1, K + 1):
        m = jnp.min(jnp.where(p > m, p, sentinel), axis=1, keepdims=True)
        mi = lax.bitcast_convert_type(m[:, 0], jnp.int32)
        idx_ref[:, t - 1] = mi & jnp.int32(N - 1)
    tab_ref[...] = jnp.concatenate(
        [R, jnp.zeros((NB, _TW - C), jnp.float32)], axis=1)


@functools.lru_cache(maxsize=None)
def _knn_call(b):
    return pl.pallas_call(
        _knn_body,
        grid=(N // NB,),
        in_specs=[
            pl.BlockSpec((1, C, N), lambda i: (b, 0, 0)),
            pl.BlockSpec((1, NB, C), lambda i: (b, i, 0)),
        ],
        out_specs=[
            pl.BlockSpec((NB, K), lambda i: (i, 0)),
            pl.BlockSpec((NB, _TW), lambda i: (i, 0)),
        ],
        out_shape=[
            jax.ShapeDtypeStruct((N, K), jnp.int32),
            jax.ShapeDtypeStruct((N, _TW), jnp.float32),
        ],
    )


_NW = 32
_NC = 2
_CHUNK = 128
_ROWS = N * K
_PER_W = _ROWS // _NW
_N_CHUNKS = _PER_W // _CHUNK


def _gather_body(table_hbm, idx_hbm, out_hbm, idx_v, rows0, rows1, sg, sw):
    wid = lax.axis_index("s") * _NC + lax.axis_index("c")
    base = pl.multiple_of(wid * _PER_W, _CHUNK)
    pltpu.sync_copy(idx_hbm.at[pl.ds(base, _PER_W)], idx_v)

    def body(i, carry):
        off0 = i * 2 * _CHUNK
        off1 = off0 + _CHUNK
        g0 = pltpu.async_copy(
            table_hbm.at[idx_v.at[pl.ds(off0, _CHUNK)]], rows0, sg)
        g1 = pltpu.async_copy(
            table_hbm.at[idx_v.at[pl.ds(off1, _CHUNK)]], rows1, sg)
        g0.wait()
        w0 = pltpu.async_copy(rows0, out_hbm.at[pl.ds(base + off0, _CHUNK)], sw)
        g1.wait()
        w1 = pltpu.async_copy(rows1, out_hbm.at[pl.ds(base + off1, _CHUNK)], sw)
        w0.wait()
        w1.wait()
        return carry

    lax.fori_loop(0, _N_CHUNKS // 2, body, 0)


@functools.lru_cache(maxsize=1)
def _gather_call():
    return pl.kernel(
        _gather_body,
        out_type=jax.ShapeDtypeStruct((_ROWS, _TW), jnp.float32),
        mesh=plsc.VectorSubcoreMesh(core_axis_name="c", subcore_axis_name="s"),
        scratch_types=[
            pltpu.VMEM((_PER_W,), jnp.int32),
            pltpu.VMEM((_CHUNK, _TW), jnp.float32),
            pltpu.VMEM((_CHUNK, _TW), jnp.float32),
            pltpu.SemaphoreType.DMA,
            pltpu.SemaphoreType.DMA,
        ],
    )


def _mlp_body(g_ref, xt_ref, x_ref, w1d_ref, b1_ref, w1b_ref, w2_ref, b2_ref,
              w3_ref, b3_ref, o_ref):
    R = xt_ref[0]
    V = lax.dot_general(R, w1d_ref[...], (((1,), (1,)), ((), ())),
                        preferred_element_type=jnp.float32) + b1_ref[...]
    G = g_ref[:, :, :C].reshape(NB2 * K, C).astype(BF)
    A1 = lax.dot_general(G, w1b_ref[...], (((1,), (1,)), ((), ())),
                         preferred_element_type=jnp.float32)
    Z1 = jnp.maximum(A1.reshape(NB2, K, CEXP) + V[:, None, :], 0.0)
    Z1 = Z1.reshape(NB2 * K, CEXP).astype(BF)
    Z2 = jnp.maximum(
        lax.dot_general(Z1, w2_ref[...], (((1,), (1,)), ((), ())),
                        preferred_element_type=jnp.float32) + b2_ref[...], 0.0)
    Z3 = lax.dot_general(Z2.astype(BF), w3_ref[...], (((1,), (1,)), ((), ())),
                         preferred_element_type=jnp.float32) + b3_ref[...]
    res = jnp.max(Z3.reshape(NB2, K, C), axis=1)
    eye = (lax.broadcasted_iota(jnp.int32, (C, C), 0) ==
           lax.broadcasted_iota(jnp.int32, (C, C), 1)).astype(jnp.float32)
    resT = lax.dot_general(eye, res, (((1,), (1,)), ((), ())),
                           preferred_element_type=jnp.float32)
    o_ref[...] = resT + x_ref[0]


@functools.lru_cache(maxsize=None)
def _mlp_call(b):
    return pl.pallas_call(
        _mlp_body,
        grid=(N // NB2,),
        in_specs=[
            pl.BlockSpec((NB2, K, _TW), lambda i: (i, 0, 0)),
            pl.BlockSpec((1, NB2, C), lambda i: (b, i, 0)),
            pl.BlockSpec((1, C, NB2), lambda i: (b, 0, i)),
            pl.BlockSpec((CEXP, C), lambda i: (0, 0)),
            pl.BlockSpec((1, CEXP), lambda i: (0, 0)),
            pl.BlockSpec((CEXP, C), lambda i: (0, 0)),
            pl.BlockSpec((CEXP, CEXP), lambda i: (0, 0)),
            pl.BlockSpec((1, CEXP), lambda i: (0, 0)),
            pl.BlockSpec((C, CEXP), lambda i: (0, 0)),
            pl.BlockSpec((1, C), lambda i: (0, 0)),
        ],
        out_specs=pl.BlockSpec((C, NB2), lambda i: (0, i)),
        out_shape=jax.ShapeDtypeStruct((C, N), jnp.float32),
    )


def kernel(input, W1, b1, W2, b2, W3, b3):
    x = input
    xt = jnp.transpose(x, (0, 2, 1))
    w1a, w1b = W1[:, :C], W1[:, C:]
    w1d = w1a - w1b
    b1r = b1.reshape(1, CEXP)
    b2r = b2.reshape(1, CEXP)
    b3r = b3.reshape(1, C)
    w1b_bf = w1b.astype(BF)
    w2_bf = W2.astype(BF)
    w3_bf = W3.astype(BF)
    knn = [_knn_call(b)(x, xt) for b in range(B)]
    gs = [_gather_call()(tab, idx.reshape(_ROWS)) for idx, tab in knn]
    outs = [_mlp_call(b)(gs[b].reshape(N, K, _TW), xt, x, w1d, b1r,
                         w1b_bf, w2_bf, b2r, w3_bf, b3r) for b in range(B)]
    return jnp.stack(outs)
